# Initial kernel scaffold; baseline (speedup 1.0000x reference)
import functools

import jax
import jax.numpy as jnp
from jax import lax
from jax.experimental import pallas as pl
from jax.experimental.pallas import tpu as pltpu
from jax.experimental.pallas import tpu_sc as plsc

N = 10000
E = 50000
D_IN = 1433
D_HID = 789
D_OUT = 7

NC = 2
NS = 16
NW = NC * NS

NPAD = 10240
DUMMY = 10100
KPAD = 1440
NCHUNK = 4
CW = 208
HPAD = NCHUNK * CW
OPAD = 16
RB = 512
ROWS_PER_TILE = NPAD // NS

EB = 128
E_PER_T16 = E // NS
NB1 = (E_PER_T16 + EB - 1) // EB
E_PER_T32 = -(-E // NW)
NB2 = (E_PER_T32 + EB - 1) // EB

_mesh = functools.partial(
    plsc.VectorSubcoreMesh, core_axis_name="c", subcore_axis_name="s")


def _deg_body(dst_hbm, zeros_hbm, out_hbm, dstv, ones_v, acc, sem):
    del sem
    cid = lax.axis_index("c")
    sid = lax.axis_index("s")
    wid = cid * NS + sid
    r0 = sid * ROWS_PER_TILE
    pltpu.sync_copy(dst_hbm.at[wid], dstv)
    pltpu.sync_copy(zeros_hbm.at[pl.ds(r0, ROWS_PER_TILE)],
                    acc.at[pl.ds(r0, ROWS_PER_TILE)])
    one_row = jnp.where(lax.iota(jnp.float32, 16) < 1.0, 1.0, 0.0)

    def fill(i, _):
        ones_v[i, :] = one_row
        return 0
    lax.fori_loop(0, EB, fill, 0)
    plsc.subcore_barrier()

    def body(j, _):
        pltpu.sync_copy(ones_v, acc.at[dstv.at[j]], add=True)
        return 0
    lax.fori_loop(0, NB2, body, 0)
    plsc.subcore_barrier()
    pltpu.sync_copy(acc.at[pl.ds(r0, ROWS_PER_TILE)],
                    out_hbm.at[cid].at[pl.ds(r0, ROWS_PER_TILE)])


def _deg_call(dst32, zeros_no):
    return pl.kernel(
        _deg_body,
        out_type=jax.ShapeDtypeStruct((NC, NPAD, OPAD), jnp.float32),
        mesh=_mesh(),
        scratch_types=[
            pltpu.VMEM((NB2, EB), jnp.int32),
            pltpu.VMEM((EB, OPAD), jnp.float32),
            pltpu.VMEM_SHARED((NPAD, OPAD), jnp.float32),
            pltpu.SemaphoreType.DMA,
        ],
    )(dst32, zeros_no)


def _agg1_body(xs_hbm, src_hbm, dst_hbm, out_hbm, srcv, dstv, gbuf, acc, sem):
    cid = lax.axis_index("c")
    sid = lax.axis_index("s")
    r0 = sid * ROWS_PER_TILE
    pltpu.sync_copy(src_hbm.at[sid], srcv)
    pltpu.sync_copy(dst_hbm.at[sid], dstv)
    for ckl in range(NCHUNK // NC):
        ck = cid * (NCHUNK // NC) + ckl
        pltpu.sync_copy(xs_hbm.at[ck].at[pl.ds(r0, ROWS_PER_TILE)],
                        acc.at[pl.ds(r0, ROWS_PER_TILE)])
        plsc.subcore_barrier()

        def body(j, _):
            pltpu.async_copy(xs_hbm.at[ck].at[srcv.at[j]], gbuf, sem).wait()
            pltpu.sync_copy(gbuf, acc.at[dstv.at[j]], add=True)
            return 0
        lax.fori_loop(0, NB1, body, 0)
        plsc.subcore_barrier()
        pltpu.sync_copy(acc.at[pl.ds(r0, ROWS_PER_TILE)],
                        out_hbm.at[ck].at[pl.ds(r0, ROWS_PER_TILE)])
        plsc.subcore_barrier()


def _agg1_call(xs, src16, dst16):
    return pl.kernel(
        _agg1_body,
        out_type=jax.ShapeDtypeStruct((NCHUNK, NPAD, CW), jnp.float32),
        mesh=_mesh(),
        scratch_types=[
            pltpu.VMEM((NB1, EB), jnp.int32),
            pltpu.VMEM((NB1, EB), jnp.int32),
            pltpu.VMEM((EB, CW), jnp.float32),
            pltpu.VMEM_SHARED((NPAD, CW), jnp.float32),
            pltpu.SemaphoreType.DMA,
        ],
    )(xs, src16, dst16)


def _agg2_body(hs_hbm, src_hbm, dst_hbm, out_hbm, srcv, dstv, gbuf, acc, sem):
    cid = lax.axis_index("c")
    sid = lax.axis_index("s")
    wid = cid * NS + sid
    r0 = sid * ROWS_PER_TILE
    pltpu.sync_copy(src_hbm.at[wid], srcv)
    pltpu.sync_copy(dst_hbm.at[wid], dstv)
    pltpu.sync_copy(hs_hbm.at[cid].at[pl.ds(r0, ROWS_PER_TILE)],
                    acc.at[pl.ds(r0, ROWS_PER_TILE)])
    plsc.subcore_barrier()

    def body(j, _):
        pltpu.async_copy(hs_hbm.at[0].at[srcv.at[j]], gbuf, sem).wait()
        pltpu.sync_copy(gbuf, acc.at[dstv.at[j]], add=True)
        return 0
    lax.fori_loop(0, NB2, body, 0)
    plsc.subcore_barrier()
    pltpu.sync_copy(acc.at[pl.ds(r0, ROWS_PER_TILE)],
                    out_hbm.at[cid].at[pl.ds(r0, ROWS_PER_TILE)])


def _agg2_call(hs2, src32, dst32):
    return pl.kernel(
        _agg2_body,
        out_type=jax.ShapeDtypeStruct((NC, NPAD, OPAD), jnp.float32),
        mesh=_mesh(),
        scratch_types=[
            pltpu.VMEM((NB2, EB), jnp.int32),
            pltpu.VMEM((NB2, EB), jnp.int32),
            pltpu.VMEM((EB, OPAD), jnp.float32),
            pltpu.VMEM_SHARED((NPAD, OPAD), jnp.float32),
            pltpu.SemaphoreType.DMA,
        ],
    )(hs2, src32, dst32)


def _dinv(degb):
    deg = degb[0][:, 0:1] + degb[1][:, 0:1] + 1.0
    return lax.rsqrt(deg)


def _mm1_body(xb, wb, degb, ob):
    dinv = _dinv(degb)
    ob[0] = jnp.dot(xb[...], wb[...],
                    preferred_element_type=jnp.float32) * dinv


def _mm1_call(xp, w1p, degp):
    return pl.pallas_call(
        _mm1_body,
        grid=(NPAD // RB, NCHUNK),
        in_specs=[
            pl.BlockSpec((RB, KPAD), lambda i, ck: (i, 0)),
            pl.BlockSpec((KPAD, CW), lambda i, ck: (0, ck)),
            pl.BlockSpec((NC, RB, OPAD), lambda i, ck: (0, i, 0)),
        ],
        out_specs=pl.BlockSpec((1, RB, CW), lambda i, ck: (ck, i, 0)),
        out_shape=jax.ShapeDtypeStruct((NCHUNK, NPAD, CW), jnp.float32),
    )(xp, w1p, degp)


def _layer2_body(aggb, degb, b1b, w2b, ob):
    dinv = _dinv(degb)
    acc = jnp.zeros((RB, OPAD), jnp.float32)
    for ck in range(NCHUNK):
        h = jnp.maximum(aggb[ck] * dinv + b1b[0:1, ck * CW:(ck + 1) * CW], 0.0)
        acc = acc + jnp.dot(h, w2b[ck * CW:(ck + 1) * CW, :],
                            preferred_element_type=jnp.float32)
    ob[0] = acc * dinv
    ob[1] = jnp.zeros((RB, OPAD), jnp.float32)


def _layer2_call(agg1, degp, b1p, w2p):
    return pl.pallas_call(
        _layer2_body,
        grid=(NPAD // RB,),
        in_specs=[
            pl.BlockSpec((NCHUNK, RB, CW), lambda i: (0, i, 0)),
            pl.BlockSpec((NC, RB, OPAD), lambda i: (0, i, 0)),
            pl.BlockSpec((1, HPAD), lambda i: (0, 0)),
            pl.BlockSpec((HPAD, OPAD), lambda i: (0, 0)),
        ],
        out_specs=pl.BlockSpec((NC, RB, OPAD), lambda i: (0, i, 0)),
        out_shape=jax.ShapeDtypeStruct((NC, NPAD, OPAD), jnp.float32),
    )(agg1, degp, b1p, w2p)


def _final_body(aggb, degb, b2b, ob):
    dinv = _dinv(degb)
    z = (aggb[0] + aggb[1]) * dinv + b2b[0:1, :]
    col = lax.broadcasted_iota(jnp.int32, (RB, OPAD), 1)
    z = jnp.where(col < D_OUT, z, -jnp.inf)
    m = jnp.max(z, axis=1, keepdims=True)
    s = jnp.sum(jnp.exp(z - m), axis=1, keepdims=True)
    ob[...] = (z - (m + jnp.log(s)))[:, :D_OUT]


def _final_call(agg2, degp, b2p):
    return pl.pallas_call(
        _final_body,
        grid=(NPAD // RB,),
        in_specs=[
            pl.BlockSpec((NC, RB, OPAD), lambda i: (0, i, 0)),
            pl.BlockSpec((NC, RB, OPAD), lambda i: (0, i, 0)),
            pl.BlockSpec((1, OPAD), lambda i: (0, 0)),
        ],
        out_specs=pl.BlockSpec((RB, D_OUT), lambda i: (i, 0)),
        out_shape=jax.ShapeDtypeStruct((NPAD, D_OUT), jnp.float32),
    )(agg2, degp, b2p)


def kernel(x, edge, W1, b1, W2, b2):
    edge = edge.astype(jnp.int32)
    src, dst = edge[0], edge[1]

    src16 = jnp.pad(src.reshape(NS, E_PER_T16),
                    ((0, 0), (0, NB1 * EB - E_PER_T16))).reshape(NS, NB1, EB)
    dst16 = jnp.pad(dst.reshape(NS, E_PER_T16),
                    ((0, 0), (0, NB1 * EB - E_PER_T16)),
                    constant_values=DUMMY).reshape(NS, NB1, EB)
    src32 = jnp.pad(src, (0, NW * NB2 * EB - E)).reshape(NW, NB2, EB)
    dst32 = jnp.pad(dst, (0, NW * NB2 * EB - E),
                    constant_values=DUMMY).reshape(NW, NB2, EB)

    xp = jnp.pad(x, ((0, NPAD - N), (0, KPAD - D_IN)))
    w1p = jnp.pad(W1, ((0, KPAD - D_IN), (0, HPAD - D_HID)))
    b1p = jnp.pad(b1, (0, HPAD - D_HID)).reshape(1, HPAD)
    w2p = jnp.pad(W2, ((0, HPAD - D_HID), (0, OPAD - D_OUT)))
    b2p = jnp.pad(b2, (0, OPAD - D_OUT)).reshape(1, OPAD)
    zeros_no = jnp.zeros((NPAD, OPAD), jnp.float32)

    degp = _deg_call(dst32, zeros_no)
    xs = _mm1_call(xp, w1p, degp)
    agg1 = _agg1_call(xs, src16, dst16)
    hs2 = _layer2_call(agg1, degp, b1p, w2p)
    agg2 = _agg2_call(hs2, src32, dst32)
    out = _final_call(agg2, degp, b2p)
    return out[:N]


# speedup vs baseline: 1.9692x; 1.9692x over previous
import functools

import jax
import jax.numpy as jnp
from jax import lax
from jax.experimental import pallas as pl
from jax.experimental.pallas import tpu as pltpu
from jax.experimental.pallas import tpu_sc as plsc

N = 10000
E = 50000
D_IN = 1433
D_HID = 789
D_OUT = 7

NC = 2
NS = 16
NW = NC * NS

NPAD = 10240
DUMMY = 10100
KPAD = 1440
NCHUNK = 6
CW = 144
HPAD = NCHUNK * CW
OPAD = 16
RB = 512
ROWS_PER_TILE = NPAD // NS

EB = 128
E_PER_T16 = E // NS
NB1 = (E_PER_T16 + EB - 1) // EB
E_PER_T32 = -(-E // NW)
NB2 = (E_PER_T32 + EB - 1) // EB

_mesh = functools.partial(
    plsc.VectorSubcoreMesh, core_axis_name="c", subcore_axis_name="s")


def _deg_body(dst_hbm, zeros_hbm, out_hbm, dstv, ones_v, acc, sem):
    del sem
    cid = lax.axis_index("c")
    sid = lax.axis_index("s")
    wid = cid * NS + sid
    r0 = sid * ROWS_PER_TILE
    pltpu.sync_copy(dst_hbm.at[wid], dstv)
    pltpu.sync_copy(zeros_hbm.at[pl.ds(r0, ROWS_PER_TILE)],
                    acc.at[pl.ds(r0, ROWS_PER_TILE)])
    one_row = jnp.where(lax.iota(jnp.int32, 16) < 1,
                        jnp.float32(1.0), jnp.float32(0.0))

    def fill(i, _):
        ones_v[i, :] = one_row
        return 0
    lax.fori_loop(0, EB, fill, 0)
    plsc.subcore_barrier()

    def body(j, _):
        pltpu.sync_copy(ones_v, acc.at[dstv.at[j]], add=True)
        return 0
    lax.fori_loop(0, NB2, body, 0)
    plsc.subcore_barrier()
    pltpu.sync_copy(acc.at[pl.ds(r0, ROWS_PER_TILE)],
                    out_hbm.at[cid].at[pl.ds(r0, ROWS_PER_TILE)])


def _deg_call(dst32, zeros_no):
    return pl.kernel(
        _deg_body,
        out_type=jax.ShapeDtypeStruct((NC, NPAD, OPAD), jnp.float32),
        mesh=_mesh(),
        compiler_params=pltpu.CompilerParams(use_tc_tiling_on_sc=False),
        scratch_types=[
            pltpu.VMEM((NB2, EB), jnp.int32),
            pltpu.VMEM((EB, OPAD), jnp.float32),
            pltpu.VMEM_SHARED((NPAD, OPAD), jnp.float32),
            pltpu.SemaphoreType.DMA,
        ],
    )(dst32, zeros_no)


def _agg1_body(xs_hbm, src_hbm, dst_hbm, out_hbm, srcv, dstv, gbuf, acc, sem):
    cid = lax.axis_index("c")
    sid = lax.axis_index("s")
    r0 = sid * ROWS_PER_TILE
    pltpu.sync_copy(src_hbm.at[sid], srcv)
    pltpu.sync_copy(dst_hbm.at[sid], dstv)
    for ckl in range(NCHUNK // NC):
        ck = cid * (NCHUNK // NC) + ckl
        pltpu.sync_copy(xs_hbm.at[ck].at[pl.ds(r0, ROWS_PER_TILE)],
                        acc.at[pl.ds(r0, ROWS_PER_TILE)])
        plsc.subcore_barrier()

        def body(j, _):
            pltpu.async_copy(xs_hbm.at[ck].at[srcv.at[j]], gbuf, sem).wait()
            pltpu.sync_copy(gbuf, acc.at[dstv.at[j]], add=True)
            return 0
        lax.fori_loop(0, NB1, body, 0)
        plsc.subcore_barrier()
        pltpu.sync_copy(acc.at[pl.ds(r0, ROWS_PER_TILE)],
                        out_hbm.at[ck].at[pl.ds(r0, ROWS_PER_TILE)])
        plsc.subcore_barrier()


def _agg1_call(xs, src16, dst16):
    return pl.kernel(
        _agg1_body,
        out_type=jax.ShapeDtypeStruct((NCHUNK, NPAD, CW), jnp.float32),
        mesh=_mesh(),
        compiler_params=pltpu.CompilerParams(use_tc_tiling_on_sc=False),
        scratch_types=[
            pltpu.VMEM((NB1, EB), jnp.int32),
            pltpu.VMEM((NB1, EB), jnp.int32),
            pltpu.VMEM((EB, CW), jnp.float32),
            pltpu.VMEM_SHARED((NPAD, CW), jnp.float32),
            pltpu.SemaphoreType.DMA,
        ],
    )(xs, src16, dst16)


def _agg2_body(hs_hbm, src_hbm, dst_hbm, out_hbm, srcv, dstv, gbuf, acc, sem):
    cid = lax.axis_index("c")
    sid = lax.axis_index("s")
    wid = cid * NS + sid
    r0 = sid * ROWS_PER_TILE
    pltpu.sync_copy(src_hbm.at[wid], srcv)
    pltpu.sync_copy(dst_hbm.at[wid], dstv)
    pltpu.sync_copy(hs_hbm.at[cid].at[pl.ds(r0, ROWS_PER_TILE)],
                    acc.at[pl.ds(r0, ROWS_PER_TILE)])
    plsc.subcore_barrier()

    def body(j, _):
        pltpu.async_copy(hs_hbm.at[0].at[srcv.at[j]], gbuf, sem).wait()
        pltpu.sync_copy(gbuf, acc.at[dstv.at[j]], add=True)
        return 0
    lax.fori_loop(0, NB2, body, 0)
    plsc.subcore_barrier()
    pltpu.sync_copy(acc.at[pl.ds(r0, ROWS_PER_TILE)],
                    out_hbm.at[cid].at[pl.ds(r0, ROWS_PER_TILE)])


def _agg2_call(hs2, src32, dst32):
    return pl.kernel(
        _agg2_body,
        out_type=jax.ShapeDtypeStruct((NC, NPAD, OPAD), jnp.float32),
        mesh=_mesh(),
        compiler_params=pltpu.CompilerParams(use_tc_tiling_on_sc=False),
        scratch_types=[
            pltpu.VMEM((NB2, EB), jnp.int32),
            pltpu.VMEM((NB2, EB), jnp.int32),
            pltpu.VMEM((EB, OPAD), jnp.float32),
            pltpu.VMEM_SHARED((NPAD, OPAD), jnp.float32),
            pltpu.SemaphoreType.DMA,
        ],
    )(hs2, src32, dst32)


def _dinv(degb):
    deg = degb[0][:, 0:1] + degb[1][:, 0:1] + 1.0
    return lax.rsqrt(deg)


def _mm1_body(xb, wb, degb, ob):
    dinv = _dinv(degb)
    ob[0] = jnp.dot(xb[...], wb[0],
                    preferred_element_type=jnp.float32) * dinv


def _mm1_call(xp, w1p, degp):
    return pl.pallas_call(
        _mm1_body,
        grid=(NPAD // RB, NCHUNK),
        in_specs=[
            pl.BlockSpec((RB, KPAD), lambda i, ck: (i, 0)),
            pl.BlockSpec((1, KPAD, CW), lambda i, ck: (ck, 0, 0)),
            pl.BlockSpec((NC, RB, OPAD), lambda i, ck: (0, i, 0)),
        ],
        out_specs=pl.BlockSpec((1, RB, CW), lambda i, ck: (ck, i, 0)),
        out_shape=jax.ShapeDtypeStruct((NCHUNK, NPAD, CW), jnp.float32),
    )(xp, w1p, degp)


def _layer2_body(aggb, degb, b1b, w2b, ob):
    dinv = _dinv(degb)
    acc = jnp.zeros((RB, OPAD), jnp.float32)
    for ck in range(NCHUNK):
        h = jnp.maximum(aggb[ck] * dinv + b1b[ck], 0.0)
        acc = acc + jnp.dot(h, w2b[ck],
                            preferred_element_type=jnp.float32)
    ob[0] = acc * dinv
    ob[1] = jnp.zeros((RB, OPAD), jnp.float32)


def _layer2_call(agg1, degp, b1p, w2p):
    return pl.pallas_call(
        _layer2_body,
        grid=(NPAD // RB,),
        in_specs=[
            pl.BlockSpec((NCHUNK, RB, CW), lambda i: (0, i, 0)),
            pl.BlockSpec((NC, RB, OPAD), lambda i: (0, i, 0)),
            pl.BlockSpec((NCHUNK, 1, CW), lambda i: (0, 0, 0)),
            pl.BlockSpec((NCHUNK, CW, OPAD), lambda i: (0, 0, 0)),
        ],
        out_specs=pl.BlockSpec((NC, RB, OPAD), lambda i: (0, i, 0)),
        out_shape=jax.ShapeDtypeStruct((NC, NPAD, OPAD), jnp.float32),
    )(agg1, degp, b1p, w2p)


def _final_body(aggb, degb, b2b, ob):
    dinv = _dinv(degb)
    z = (aggb[0] + aggb[1]) * dinv + b2b[0:1, :]
    col = lax.broadcasted_iota(jnp.int32, (RB, OPAD), 1)
    z = jnp.where(col < D_OUT, z, -jnp.inf)
    m = jnp.max(z, axis=1, keepdims=True)
    s = jnp.sum(jnp.exp(z - m), axis=1, keepdims=True)
    ob[...] = (z - (m + jnp.log(s)))[:, :D_OUT]


def _final_call(agg2, degp, b2p):
    return pl.pallas_call(
        _final_body,
        grid=(NPAD // RB,),
        in_specs=[
            pl.BlockSpec((NC, RB, OPAD), lambda i: (0, i, 0)),
            pl.BlockSpec((NC, RB, OPAD), lambda i: (0, i, 0)),
            pl.BlockSpec((1, OPAD), lambda i: (0, 0)),
        ],
        out_specs=pl.BlockSpec((RB, D_OUT), lambda i: (i, 0)),
        out_shape=jax.ShapeDtypeStruct((NPAD, D_OUT), jnp.float32),
    )(agg2, degp, b2p)


def kernel(x, edge, W1, b1, W2, b2):
    edge = edge.astype(jnp.int32)
    src, dst = edge[0], edge[1]

    src16 = jnp.pad(src.reshape(NS, E_PER_T16),
                    ((0, 0), (0, NB1 * EB - E_PER_T16))).reshape(NS, NB1, EB)
    dst16 = jnp.pad(dst.reshape(NS, E_PER_T16),
                    ((0, 0), (0, NB1 * EB - E_PER_T16)),
                    constant_values=DUMMY).reshape(NS, NB1, EB)
    src32 = jnp.pad(src, (0, NW * NB2 * EB - E)).reshape(NW, NB2, EB)
    dst32 = jnp.pad(dst, (0, NW * NB2 * EB - E),
                    constant_values=DUMMY).reshape(NW, NB2, EB)

    xp = jnp.pad(x, ((0, NPAD - N), (0, KPAD - D_IN)))
    w1p = jnp.pad(W1, ((0, KPAD - D_IN), (0, HPAD - D_HID)))
    w1p = w1p.reshape(KPAD, NCHUNK, CW).transpose(1, 0, 2)
    b1p = jnp.pad(b1, (0, HPAD - D_HID)).reshape(NCHUNK, 1, CW)
    w2p = jnp.pad(W2, ((0, HPAD - D_HID), (0, OPAD - D_OUT)))
    w2p = w2p.reshape(NCHUNK, CW, OPAD)
    b2p = jnp.pad(b2, (0, OPAD - D_OUT)).reshape(1, OPAD)
    zeros_no = jnp.zeros((NPAD, OPAD), jnp.float32)

    degp = _deg_call(dst32, zeros_no)
    xs = _mm1_call(xp, w1p, degp)
    agg1 = _agg1_call(xs, src16, dst16)
    hs2 = _layer2_call(agg1, degp, b1p, w2p)
    agg2 = _agg2_call(hs2, src32, dst32)
    out = _final_call(agg2, degp, b2p)
    return out[:N]


# speedup vs baseline: 2.1480x; 1.0908x over previous
import functools

import jax
import jax.numpy as jnp
from jax import lax
from jax.experimental import pallas as pl
from jax.experimental.pallas import tpu as pltpu
from jax.experimental.pallas import tpu_sc as plsc

N = 10000
E = 50000
D_IN = 1433
D_HID = 789
D_OUT = 7

NC = 2
NS = 16
NW = NC * NS

NPAD = 10240
DUMMY = 10100
NCHUNK = 7
CW = 128
CPC = 4
HPAD = NCHUNK * CW
OPAD = 16
RB = 512
ROWS_PER_TILE = NPAD // NS

EB = 128
E_PER_T16 = E // NS
NB1 = 26
E_PER_T32 = -(-E // NW)
NB2 = (E_PER_T32 + EB - 1) // EB

_mesh = functools.partial(
    plsc.VectorSubcoreMesh, core_axis_name="c", subcore_axis_name="s")


def _deg_body(dst_hbm, zeros_hbm, out_hbm, dstv, ones_v, acc, sem):
    del sem
    cid = lax.axis_index("c")
    sid = lax.axis_index("s")
    wid = cid * NS + sid
    r0 = sid * ROWS_PER_TILE
    pltpu.sync_copy(dst_hbm.at[wid], dstv)
    pltpu.sync_copy(zeros_hbm.at[pl.ds(r0, ROWS_PER_TILE)],
                    acc.at[pl.ds(r0, ROWS_PER_TILE)])
    one_row = jnp.where(lax.iota(jnp.int32, 16) < 1,
                        jnp.float32(1.0), jnp.float32(0.0))

    def fill(i, _):
        ones_v[i, :] = one_row
        return 0
    lax.fori_loop(0, EB, fill, 0)
    plsc.subcore_barrier()

    def body(j, _):
        pltpu.sync_copy(ones_v, acc.at[dstv.at[j]], add=True)
        return 0
    lax.fori_loop(0, NB2, body, 0)
    plsc.subcore_barrier()
    pltpu.sync_copy(acc.at[pl.ds(r0, ROWS_PER_TILE)],
                    out_hbm.at[cid].at[pl.ds(r0, ROWS_PER_TILE)])


def _deg_call(dst32, zeros_no):
    return pl.kernel(
        _deg_body,
        out_type=jax.ShapeDtypeStruct((NC, NPAD, OPAD), jnp.float32),
        mesh=_mesh(),
        compiler_params=pltpu.CompilerParams(use_tc_tiling_on_sc=False),
        scratch_types=[
            pltpu.VMEM((NB2, EB), jnp.int32),
            pltpu.VMEM((EB, OPAD), jnp.float32),
            pltpu.VMEM_SHARED((NPAD, OPAD), jnp.float32),
            pltpu.SemaphoreType.DMA,
        ],
    )(dst32, zeros_no)


def _agg1_body(xs_hbm, src_hbm, dst_hbm, out_hbm, srcv, dstv,
               gbuf0, gbuf1, acc, sem0, sem1):
    cid = lax.axis_index("c")
    sid = lax.axis_index("s")
    r0 = sid * ROWS_PER_TILE
    pltpu.sync_copy(src_hbm.at[sid], srcv)
    pltpu.sync_copy(dst_hbm.at[sid], dstv)
    bufs = (gbuf0, gbuf1)
    sems = (sem0, sem1)
    for ckl in range(CPC):
        ck = cid * CPC + ckl
        valid = ck < NCHUNK

        @pl.when(valid)
        def _init():
            pltpu.sync_copy(xs_hbm.at[ck].at[pl.ds(r0, ROWS_PER_TILE)],
                            acc.at[pl.ds(r0, ROWS_PER_TILE)])
        plsc.subcore_barrier()

        @pl.when(valid)
        def _edges():
            pltpu.async_copy(xs_hbm.at[ck].at[srcv.at[0]], bufs[0], sems[0])
            for b in range(NB1):
                if b + 1 < NB1:
                    pltpu.async_copy(xs_hbm.at[ck].at[srcv.at[b + 1]],
                                     bufs[(b + 1) % 2], sems[(b + 1) % 2])
                pltpu.make_async_copy(xs_hbm.at[ck].at[srcv.at[b]],
                                      bufs[b % 2], sems[b % 2]).wait()
                pltpu.sync_copy(bufs[b % 2], acc.at[dstv.at[b]], add=True)
        plsc.subcore_barrier()

        @pl.when(valid)
        def _writeout():
            pltpu.sync_copy(acc.at[pl.ds(r0, ROWS_PER_TILE)],
                            out_hbm.at[ck].at[pl.ds(r0, ROWS_PER_TILE)])
        plsc.subcore_barrier()


def _agg1_call(xs, src16, dst16):
    return pl.kernel(
        _agg1_body,
        out_type=jax.ShapeDtypeStruct((NCHUNK, NPAD, CW), jnp.float32),
        mesh=_mesh(),
        compiler_params=pltpu.CompilerParams(use_tc_tiling_on_sc=True),
        scratch_types=[
            pltpu.VMEM((NB1, EB), jnp.int32),
            pltpu.VMEM((NB1, EB), jnp.int32),
            pltpu.VMEM((EB, CW), jnp.float32),
            pltpu.VMEM((EB, CW), jnp.float32),
            pltpu.VMEM_SHARED((NPAD, CW), jnp.float32),
            pltpu.SemaphoreType.DMA,
            pltpu.SemaphoreType.DMA,
        ],
    )(xs, src16, dst16)


def _agg2_body(hs_hbm, src_hbm, dst_hbm, out_hbm, srcv, dstv, gbuf, acc, sem):
    cid = lax.axis_index("c")
    sid = lax.axis_index("s")
    wid = cid * NS + sid
    r0 = sid * ROWS_PER_TILE
    pltpu.sync_copy(src_hbm.at[wid], srcv)
    pltpu.sync_copy(dst_hbm.at[wid], dstv)
    pltpu.sync_copy(hs_hbm.at[cid].at[pl.ds(r0, ROWS_PER_TILE)],
                    acc.at[pl.ds(r0, ROWS_PER_TILE)])
    plsc.subcore_barrier()

    def body(j, _):
        pltpu.async_copy(hs_hbm.at[0].at[srcv.at[j]], gbuf, sem).wait()
        pltpu.sync_copy(gbuf, acc.at[dstv.at[j]], add=True)
        return 0
    lax.fori_loop(0, NB2, body, 0)
    plsc.subcore_barrier()
    pltpu.sync_copy(acc.at[pl.ds(r0, ROWS_PER_TILE)],
                    out_hbm.at[cid].at[pl.ds(r0, ROWS_PER_TILE)])


def _agg2_call(hs2, src32, dst32):
    return pl.kernel(
        _agg2_body,
        out_type=jax.ShapeDtypeStruct((NC, NPAD, OPAD), jnp.float32),
        mesh=_mesh(),
        compiler_params=pltpu.CompilerParams(use_tc_tiling_on_sc=False),
        scratch_types=[
            pltpu.VMEM((NB2, EB), jnp.int32),
            pltpu.VMEM((NB2, EB), jnp.int32),
            pltpu.VMEM((EB, OPAD), jnp.float32),
            pltpu.VMEM_SHARED((NPAD, OPAD), jnp.float32),
            pltpu.SemaphoreType.DMA,
        ],
    )(hs2, src32, dst32)


def _dinv(degb):
    deg = degb[0][:, 0:1] + degb[1][:, 0:1] + 1.0
    return lax.rsqrt(deg)


def _mm1_body(xb, wb, degb, ob):
    dinv = _dinv(degb)
    ob[0] = jnp.dot(xb[...], wb[0],
                    preferred_element_type=jnp.float32) * dinv


def _mm1_call(xp, w1p, degp):
    return pl.pallas_call(
        _mm1_body,
        grid=(NPAD // RB, NCHUNK),
        in_specs=[
            pl.BlockSpec((RB, D_IN), lambda i, ck: (i, 0)),
            pl.BlockSpec((1, D_IN, CW), lambda i, ck: (ck, 0, 0)),
            pl.BlockSpec((NC, RB, OPAD), lambda i, ck: (0, i, 0)),
        ],
        out_specs=pl.BlockSpec((1, RB, CW), lambda i, ck: (ck, i, 0)),
        out_shape=jax.ShapeDtypeStruct((NCHUNK, NPAD, CW), jnp.float32),
    )(xp, w1p, degp)


def _layer2_body(aggb, degb, b1b, w2b, ob):
    dinv = _dinv(degb)
    acc = jnp.zeros((RB, OPAD), jnp.float32)
    for ck in range(NCHUNK):
        h = jnp.maximum(aggb[ck] * dinv + b1b[ck], 0.0)
        acc = acc + jnp.dot(h, w2b[ck],
                            preferred_element_type=jnp.float32)
    ob[0] = acc * dinv
    ob[1] = jnp.zeros((RB, OPAD), jnp.float32)


def _layer2_call(agg1, degp, b1p, w2p):
    return pl.pallas_call(
        _layer2_body,
        grid=(NPAD // RB,),
        in_specs=[
            pl.BlockSpec((NCHUNK, RB, CW), lambda i: (0, i, 0)),
            pl.BlockSpec((NC, RB, OPAD), lambda i: (0, i, 0)),
            pl.BlockSpec((NCHUNK, 1, CW), lambda i: (0, 0, 0)),
            pl.BlockSpec((NCHUNK, CW, OPAD), lambda i: (0, 0, 0)),
        ],
        out_specs=pl.BlockSpec((NC, RB, OPAD), lambda i: (0, i, 0)),
        out_shape=jax.ShapeDtypeStruct((NC, NPAD, OPAD), jnp.float32),
    )(agg1, degp, b1p, w2p)


def _final_body(aggb, degb, b2b, ob):
    dinv = _dinv(degb)
    z = (aggb[0] + aggb[1]) * dinv + b2b[0:1, :]
    col = lax.broadcasted_iota(jnp.int32, (RB, OPAD), 1)
    z = jnp.where(col < D_OUT, z, -jnp.inf)
    m = jnp.max(z, axis=1, keepdims=True)
    s = jnp.sum(jnp.exp(z - m), axis=1, keepdims=True)
    ob[...] = (z - (m + jnp.log(s)))[:, :D_OUT]


def _final_call(agg2, degp, b2p):
    return pl.pallas_call(
        _final_body,
        grid=(NPAD // RB,),
        in_specs=[
            pl.BlockSpec((NC, RB, OPAD), lambda i: (0, i, 0)),
            pl.BlockSpec((NC, RB, OPAD), lambda i: (0, i, 0)),
            pl.BlockSpec((1, OPAD), lambda i: (0, 0)),
        ],
        out_specs=pl.BlockSpec((RB, D_OUT), lambda i: (i, 0)),
        out_shape=jax.ShapeDtypeStruct((NPAD, D_OUT), jnp.float32),
    )(agg2, degp, b2p)


def kernel(x, edge, W1, b1, W2, b2):
    edge = edge.astype(jnp.int32)
    src, dst = edge[0], edge[1]

    src16 = jnp.pad(src.reshape(NS, E_PER_T16),
                    ((0, 0), (0, NB1 * EB - E_PER_T16))).reshape(NS, NB1, EB)
    dst16 = jnp.pad(dst.reshape(NS, E_PER_T16),
                    ((0, 0), (0, NB1 * EB - E_PER_T16)),
                    constant_values=DUMMY).reshape(NS, NB1, EB)
    src32 = jnp.pad(src, (0, NW * NB2 * EB - E)).reshape(NW, NB2, EB)
    dst32 = jnp.pad(dst, (0, NW * NB2 * EB - E),
                    constant_values=DUMMY).reshape(NW, NB2, EB)

    w1p = jnp.pad(W1, ((0, 0), (0, HPAD - D_HID)))
    w1p = w1p.reshape(D_IN, NCHUNK, CW).transpose(1, 0, 2)
    b1p = jnp.pad(b1, (0, HPAD - D_HID)).reshape(NCHUNK, 1, CW)
    w2p = jnp.pad(W2, ((0, HPAD - D_HID), (0, OPAD - D_OUT)))
    w2p = w2p.reshape(NCHUNK, CW, OPAD)
    b2p = jnp.pad(b2, (0, OPAD - D_OUT)).reshape(1, OPAD)
    zeros_no = jnp.zeros((NPAD, OPAD), jnp.float32)

    degp = _deg_call(dst32, zeros_no)
    xs = _mm1_call(x, w1p, degp)
    agg1 = _agg1_call(xs, src16, dst16)
    hs2 = _layer2_call(agg1, degp, b1p, w2p)
    agg2 = _agg2_call(hs2, src32, dst32)
    out = _final_call(agg2, degp, b2p)
    return out[:N]


# speedup vs baseline: 3.0223x; 1.4071x over previous
import functools

import jax
import jax.numpy as jnp
from jax import lax
from jax.experimental import pallas as pl
from jax.experimental.pallas import tpu as pltpu
from jax.experimental.pallas import tpu_sc as plsc

N = 10000
E = 50000
D_IN = 1433
D_HID = 789
D_OUT = 7

NC = 2
NS = 16

NPAD = 10240
DUMMY = 10100
NCHUNK = 7
CW = 128
HPAD = NCHUNK * CW
OPAD = 16
RB = 512
ROWS_PER_TILE = NPAD // NS

EB = 128
E_PER_T = E // NS
NB = 26
NBH = NB // 2
NBUF = 4
G = 2

_mesh = functools.partial(
    plsc.VectorSubcoreMesh, core_axis_name="c", subcore_axis_name="s")


def _fmt_body(eb, ob):
    for c in range(2):
        pad_val = 0 if c == 0 else DUMMY
        row = eb[c]
        fill = jnp.full((NB * EB - E_PER_T,), pad_val, jnp.int32)
        for t in range(NS):
            seg = jnp.concatenate([row[t * E_PER_T:(t + 1) * E_PER_T], fill])
            ob[c, t] = seg.reshape(NB, EB)


def _fmt_call(edge):
    return pl.pallas_call(
        _fmt_body,
        grid=(1,),
        in_specs=[pl.BlockSpec((2, E), lambda i: (0, 0))],
        out_specs=pl.BlockSpec((2, NS, NB, EB), lambda i: (0, 0, 0, 0)),
        out_shape=jax.ShapeDtypeStruct((2, NS, NB, EB), jnp.int32),
    )(edge)


def _deg_body(idx_hbm, zeros_hbm, out_hbm, dstv, ones_v, acc, sem):
    del sem
    cid = lax.axis_index("c")
    sid = lax.axis_index("s")
    r0 = sid * ROWS_PER_TILE
    pltpu.sync_copy(idx_hbm.at[1].at[sid], dstv)
    pltpu.sync_copy(zeros_hbm.at[pl.ds(r0, ROWS_PER_TILE)],
                    acc.at[pl.ds(r0, ROWS_PER_TILE)])
    one_row = jnp.where(lax.iota(jnp.int32, 16) < 1,
                        jnp.float32(1.0), jnp.float32(0.0))

    def fill(i, _):
        ones_v[i, :] = one_row
        return 0
    lax.fori_loop(0, EB, fill, 0)
    plsc.subcore_barrier()

    def body(j, _):
        pltpu.sync_copy(ones_v, acc.at[dstv.at[2 * j + cid]], add=True)
        return 0
    lax.fori_loop(0, NBH, body, 0)
    plsc.subcore_barrier()
    pltpu.sync_copy(acc.at[pl.ds(r0, ROWS_PER_TILE)],
                    out_hbm.at[cid].at[pl.ds(r0, ROWS_PER_TILE)])


def _deg_call(idx16, zeros_no):
    return pl.kernel(
        _deg_body,
        out_type=jax.ShapeDtypeStruct((NC, NPAD, OPAD), jnp.float32),
        mesh=_mesh(),
        compiler_params=pltpu.CompilerParams(use_tc_tiling_on_sc=False),
        scratch_types=[
            pltpu.VMEM((NB, EB), jnp.int32),
            pltpu.VMEM((EB, OPAD), jnp.float32),
            pltpu.VMEM_SHARED((NPAD, OPAD), jnp.float32),
            pltpu.SemaphoreType.DMA,
        ],
    )(idx16, zeros_no)


def _ring(xs_chunk, srcv, dstv, acc, bufs, gsems, ssems, b0, nb):
    for g in range(G):
        pltpu.async_copy(xs_chunk.at[srcv.at[b0 + g]],
                         bufs[g % NBUF], gsems[g % NBUF])
    for b in range(nb):
        g = b + G
        if g < nb:
            if g - NBUF >= 0:
                pltpu.make_async_copy(bufs[g % NBUF],
                                      acc.at[dstv.at[b0 + g - NBUF]],
                                      ssems[g % NBUF]).wait()
            pltpu.async_copy(xs_chunk.at[srcv.at[b0 + g]],
                             bufs[g % NBUF], gsems[g % NBUF])
        pltpu.make_async_copy(xs_chunk.at[srcv.at[b0 + b]],
                              bufs[b % NBUF], gsems[b % NBUF]).wait()
        pltpu.async_copy(bufs[b % NBUF], acc.at[dstv.at[b0 + b]],
                         ssems[b % NBUF], add=True)
    for b in range(max(0, nb - NBUF), nb):
        pltpu.make_async_copy(bufs[b % NBUF], acc.at[dstv.at[b0 + b]],
                              ssems[b % NBUF]).wait()


def _agg1_body(xs_hbm, idx_hbm, out_hbm, srcv, dstv,
               bufs, acc, gsems, ssems):
    cid = lax.axis_index("c")
    sid = lax.axis_index("s")
    r0 = sid * ROWS_PER_TILE
    pltpu.sync_copy(idx_hbm.at[0].at[sid], srcv)
    pltpu.sync_copy(idx_hbm.at[1].at[sid], dstv)
    for ckl in range(NCHUNK // NC):
        ck = cid * (NCHUNK // NC) + ckl
        pltpu.sync_copy(xs_hbm.at[ck].at[pl.ds(r0, ROWS_PER_TILE)],
                        acc.at[pl.ds(r0, ROWS_PER_TILE)])
        plsc.subcore_barrier()
        _ring(xs_hbm.at[ck], srcv, dstv, acc, bufs, gsems, ssems, 0, NB)
        plsc.subcore_barrier()
        pltpu.sync_copy(acc.at[pl.ds(r0, ROWS_PER_TILE)],
                        out_hbm.at[ck].at[pl.ds(r0, ROWS_PER_TILE)])
        plsc.subcore_barrier()

    @pl.when(cid == 0)
    def _seed_self():
        pltpu.sync_copy(xs_hbm.at[NCHUNK - 1].at[pl.ds(r0, ROWS_PER_TILE)],
                        acc.at[pl.ds(r0, ROWS_PER_TILE)])

    @pl.when(cid == 1)
    def _seed_zero():
        zrow = jnp.zeros((32,), jnp.bfloat16)

        def zfill(i, _):
            for k in range(CW // 32):
                bufs[0][i, pl.ds(k * 32, 32)] = zrow
            return 0
        lax.fori_loop(0, EB, zfill, 0)
        for i in range(ROWS_PER_TILE // EB):
            pltpu.sync_copy(bufs[0], acc.at[pl.ds(r0 + i * EB, EB)])
    plsc.subcore_barrier()
    _ring(xs_hbm.at[NCHUNK - 1], srcv, dstv, acc, bufs, gsems, ssems,
          cid * NBH, NBH)
    plsc.subcore_barrier()
    pltpu.sync_copy(acc.at[pl.ds(r0, ROWS_PER_TILE)],
                    out_hbm.at[NCHUNK - 1 + cid].at[pl.ds(r0, ROWS_PER_TILE)])


def _agg1_call(xs, idx16):
    return pl.kernel(
        _agg1_body,
        out_type=jax.ShapeDtypeStruct((NCHUNK + 1, NPAD, CW), jnp.bfloat16),
        mesh=_mesh(),
        compiler_params=pltpu.CompilerParams(use_tc_tiling_on_sc=False),
        scratch_types=[
            pltpu.VMEM((NB, EB), jnp.int32),
            pltpu.VMEM((NB, EB), jnp.int32),
            [pltpu.VMEM((EB, CW), jnp.bfloat16) for _ in range(NBUF)],
            pltpu.VMEM_SHARED((NPAD, CW), jnp.bfloat16),
            [pltpu.SemaphoreType.DMA for _ in range(NBUF)],
            [pltpu.SemaphoreType.DMA for _ in range(NBUF)],
        ],
    )(xs, idx16)


def _agg2_body(hs_hbm, idx_hbm, out_hbm, srcv, dstv, gbuf, acc, sem):
    cid = lax.axis_index("c")
    sid = lax.axis_index("s")
    r0 = sid * ROWS_PER_TILE
    pltpu.sync_copy(idx_hbm.at[0].at[sid], srcv)
    pltpu.sync_copy(idx_hbm.at[1].at[sid], dstv)
    pltpu.sync_copy(hs_hbm.at[cid].at[pl.ds(r0, ROWS_PER_TILE)],
                    acc.at[pl.ds(r0, ROWS_PER_TILE)])
    plsc.subcore_barrier()

    def body(j, _):
        b = 2 * j + cid
        pltpu.async_copy(hs_hbm.at[0].at[srcv.at[b]], gbuf, sem).wait()
        pltpu.sync_copy(gbuf, acc.at[dstv.at[b]], add=True)
        return 0
    lax.fori_loop(0, NBH, body, 0)
    plsc.subcore_barrier()
    pltpu.sync_copy(acc.at[pl.ds(r0, ROWS_PER_TILE)],
                    out_hbm.at[cid].at[pl.ds(r0, ROWS_PER_TILE)])


def _agg2_call(hs2, idx16):
    return pl.kernel(
        _agg2_body,
        out_type=jax.ShapeDtypeStruct((NC, NPAD, OPAD), jnp.float32),
        mesh=_mesh(),
        compiler_params=pltpu.CompilerParams(use_tc_tiling_on_sc=False),
        scratch_types=[
            pltpu.VMEM((NB, EB), jnp.int32),
            pltpu.VMEM((NB, EB), jnp.int32),
            pltpu.VMEM((EB, OPAD), jnp.float32),
            pltpu.VMEM_SHARED((NPAD, OPAD), jnp.float32),
            pltpu.SemaphoreType.DMA,
        ],
    )(hs2, idx16)


def _dinv(degb):
    deg = degb[0][:, 0:1] + degb[1][:, 0:1] + 1.0
    return lax.rsqrt(deg)


def _mm1_body(xb, wb, degb, ob):
    dinv = _dinv(degb)
    xbf = xb[...].astype(jnp.bfloat16)
    ys = (jnp.dot(xbf, wb[...], preferred_element_type=jnp.float32)
          * dinv).astype(jnp.bfloat16)
    for ck in range(NCHUNK):
        ob[ck] = ys[:, ck * CW:(ck + 1) * CW]


def _mm1_call(xp, w1p, degp):
    return pl.pallas_call(
        _mm1_body,
        grid=(NPAD // RB,),
        in_specs=[
            pl.BlockSpec((RB, D_IN), lambda i: (i, 0)),
            pl.BlockSpec((D_IN, HPAD), lambda i: (0, 0)),
            pl.BlockSpec((NC, RB, OPAD), lambda i: (0, i, 0)),
        ],
        out_specs=pl.BlockSpec((NCHUNK, RB, CW), lambda i: (0, i, 0)),
        out_shape=jax.ShapeDtypeStruct((NCHUNK, NPAD, CW), jnp.bfloat16),
    )(xp, w1p, degp)


def _layer2_body(aggb, degb, b1b, w2b, ob):
    dinv = _dinv(degb)
    acc = jnp.zeros((RB, OPAD), jnp.float32)
    for ck in range(NCHUNK):
        a = aggb[ck].astype(jnp.float32)
        if ck == NCHUNK - 1:
            a = a + aggb[NCHUNK].astype(jnp.float32)
        h = jnp.maximum(a * dinv + b1b[ck], 0.0)
        acc = acc + jnp.dot(h, w2b[ck],
                            preferred_element_type=jnp.float32)
    ob[0] = acc * dinv
    ob[1] = jnp.zeros((RB, OPAD), jnp.float32)


def _layer2_call(agg1, degp, b1p, w2p):
    return pl.pallas_call(
        _layer2_body,
        grid=(NPAD // RB,),
        in_specs=[
            pl.BlockSpec((NCHUNK + 1, RB, CW), lambda i: (0, i, 0)),
            pl.BlockSpec((NC, RB, OPAD), lambda i: (0, i, 0)),
            pl.BlockSpec((NCHUNK, 1, CW), lambda i: (0, 0, 0)),
            pl.BlockSpec((NCHUNK, CW, OPAD), lambda i: (0, 0, 0)),
        ],
        out_specs=pl.BlockSpec((NC, RB, OPAD), lambda i: (0, i, 0)),
        out_shape=jax.ShapeDtypeStruct((NC, NPAD, OPAD), jnp.float32),
    )(agg1, degp, b1p, w2p)


def _final_body(aggb, degb, b2b, ob):
    dinv = _dinv(degb)
    z = (aggb[0] + aggb[1]) * dinv + b2b[0:1, :]
    col = lax.broadcasted_iota(jnp.int32, (RB, OPAD), 1)
    z = jnp.where(col < D_OUT, z, -jnp.inf)
    m = jnp.max(z, axis=1, keepdims=True)
    s = jnp.sum(jnp.exp(z - m), axis=1, keepdims=True)
    ob[...] = (z - (m + jnp.log(s)))[:, :D_OUT]


def _final_call(agg2, degp, b2p):
    return pl.pallas_call(
        _final_body,
        grid=(NPAD // RB,),
        in_specs=[
            pl.BlockSpec((NC, RB, OPAD), lambda i: (0, i, 0)),
            pl.BlockSpec((NC, RB, OPAD), lambda i: (0, i, 0)),
            pl.BlockSpec((1, OPAD), lambda i: (0, 0)),
        ],
        out_specs=pl.BlockSpec((RB, D_OUT), lambda i: (i, 0)),
        out_shape=jax.ShapeDtypeStruct((NPAD, D_OUT), jnp.float32),
    )(agg2, degp, b2p)


def kernel(x, edge, W1, b1, W2, b2):
    edge = edge.astype(jnp.int32)
    idx16 = _fmt_call(edge)

    w1p = jnp.pad(W1, ((0, 0), (0, HPAD - D_HID))).astype(jnp.bfloat16)
    b1p = jnp.pad(b1, (0, HPAD - D_HID)).reshape(NCHUNK, 1, CW)
    w2p = jnp.pad(W2, ((0, HPAD - D_HID), (0, OPAD - D_OUT)))
    w2p = w2p.reshape(NCHUNK, CW, OPAD)
    b2p = jnp.pad(b2, (0, OPAD - D_OUT)).reshape(1, OPAD)
    zeros_no = jnp.zeros((NPAD, OPAD), jnp.float32)

    degp = _deg_call(idx16, zeros_no)
    xs = _mm1_call(x, w1p, degp)
    agg1 = _agg1_call(xs, idx16)
    hs2 = _layer2_call(agg1, degp, b1p, w2p)
    agg2 = _agg2_call(hs2, idx16)
    out = _final_call(agg2, degp, b2p)
    return out[:N]


# speedup vs baseline: 3.1132x; 1.0301x over previous
import functools

import jax
import jax.numpy as jnp
from jax import lax
from jax.experimental import pallas as pl
from jax.experimental.pallas import tpu as pltpu
from jax.experimental.pallas import tpu_sc as plsc

N = 10000
E = 50000
D_IN = 1433
D_HID = 789
D_OUT = 7

NC = 2
NS = 16

NPAD = 10240
DUMMY = 10100
NCHUNK = 4
CW = 224
HPAD = NCHUNK * CW
OPAD = 16
RB = 512
ROWS_PER_TILE = NPAD // NS

EB = 128
E_PER_T = E // NS
NB = 26
NBR = 32
NBH = NB // 2
NBUF = 3
G = 2

_mesh = functools.partial(
    plsc.VectorSubcoreMesh, core_axis_name="c", subcore_axis_name="s")


def _fmt_body(eb, ob):
    for c in range(2):
        pad_val = 0 if c == 0 else DUMMY
        row = eb[c]
        fill = jnp.full((NBR * EB - E_PER_T,), pad_val, jnp.int32)
        for t in range(NS):
            seg = jnp.concatenate([row[t * E_PER_T:(t + 1) * E_PER_T], fill])
            ob[c, t] = seg.reshape(NBR, EB)


def _fmt_call(edge):
    return pl.pallas_call(
        _fmt_body,
        grid=(1,),
        in_specs=[pl.BlockSpec((2, E), lambda i: (0, 0))],
        out_specs=pl.BlockSpec((2, NS, NBR, EB), lambda i: (0, 0, 0, 0)),
        out_shape=jax.ShapeDtypeStruct((2, NS, NBR, EB), jnp.int32),
    )(edge)


def _deg_body(idx_hbm, zeros_hbm, out_hbm, dstv, ones_v, acc, sem):
    del sem
    cid = lax.axis_index("c")
    sid = lax.axis_index("s")
    r0 = sid * ROWS_PER_TILE
    pltpu.sync_copy(idx_hbm.at[1].at[sid].at[pl.ds(0, NB)], dstv)
    pltpu.sync_copy(zeros_hbm.at[pl.ds(r0, ROWS_PER_TILE)],
                    acc.at[pl.ds(r0, ROWS_PER_TILE)])
    one_row = jnp.where(lax.iota(jnp.int32, 16) < 1,
                        jnp.float32(1.0), jnp.float32(0.0))

    def fill(i, _):
        ones_v[i, :] = one_row
        return 0
    lax.fori_loop(0, EB, fill, 0)
    plsc.subcore_barrier()

    def body(j, _):
        pltpu.sync_copy(ones_v, acc.at[dstv.at[2 * j + cid]], add=True)
        return 0
    lax.fori_loop(0, NBH, body, 0)
    plsc.subcore_barrier()
    pltpu.sync_copy(acc.at[pl.ds(r0, ROWS_PER_TILE)],
                    out_hbm.at[cid].at[pl.ds(r0, ROWS_PER_TILE)])


def _deg_call(idx16, zeros_no):
    return pl.kernel(
        _deg_body,
        out_type=jax.ShapeDtypeStruct((NC, NPAD, OPAD), jnp.float32),
        mesh=_mesh(),
        compiler_params=pltpu.CompilerParams(use_tc_tiling_on_sc=False),
        scratch_types=[
            pltpu.VMEM((NB, EB), jnp.int32),
            pltpu.VMEM((EB, OPAD), jnp.float32),
            pltpu.VMEM_SHARED((NPAD, OPAD), jnp.float32),
            pltpu.SemaphoreType.DMA,
        ],
    )(idx16, zeros_no)


def _ring(xs_chunk, srcv, dstv, acc, bufs, gsems, ssems, b0, nb):
    for g in range(G):
        pltpu.async_copy(xs_chunk.at[srcv.at[b0 + g]],
                         bufs[g % NBUF], gsems[g % NBUF])
    for b in range(nb):
        g = b + G
        if g < nb:
            if g - NBUF >= 0:
                pltpu.make_async_copy(bufs[g % NBUF],
                                      acc.at[dstv.at[b0 + g - NBUF]],
                                      ssems[g % NBUF]).wait()
            pltpu.async_copy(xs_chunk.at[srcv.at[b0 + g]],
                             bufs[g % NBUF], gsems[g % NBUF])
        pltpu.make_async_copy(xs_chunk.at[srcv.at[b0 + b]],
                              bufs[b % NBUF], gsems[b % NBUF]).wait()
        pltpu.async_copy(bufs[b % NBUF], acc.at[dstv.at[b0 + b]],
                         ssems[b % NBUF], add=True)
    for b in range(max(0, nb - NBUF), nb):
        pltpu.make_async_copy(bufs[b % NBUF], acc.at[dstv.at[b0 + b]],
                              ssems[b % NBUF]).wait()


def _agg1_body(xs_hbm, idx_hbm, out_hbm, srcv, dstv,
               bufs, acc, gsems, ssems):
    cid = lax.axis_index("c")
    sid = lax.axis_index("s")
    r0 = sid * ROWS_PER_TILE
    pltpu.sync_copy(idx_hbm.at[0].at[sid].at[pl.ds(0, NB)], srcv)
    pltpu.sync_copy(idx_hbm.at[1].at[sid].at[pl.ds(0, NB)], dstv)
    for ckl in range(NCHUNK // NC):
        ck = cid * (NCHUNK // NC) + ckl
        pltpu.sync_copy(xs_hbm.at[ck].at[pl.ds(r0, ROWS_PER_TILE)],
                        acc.at[pl.ds(r0, ROWS_PER_TILE)])
        plsc.subcore_barrier()
        _ring(xs_hbm.at[ck], srcv, dstv, acc, bufs, gsems, ssems, 0, NB)
        plsc.subcore_barrier()
        pltpu.sync_copy(acc.at[pl.ds(r0, ROWS_PER_TILE)],
                        out_hbm.at[ck].at[pl.ds(r0, ROWS_PER_TILE)])
        plsc.subcore_barrier()


def _agg1_call(xs, idx16):
    return pl.kernel(
        _agg1_body,
        out_type=jax.ShapeDtypeStruct((NCHUNK, NPAD, CW), jnp.bfloat16),
        mesh=_mesh(),
        compiler_params=pltpu.CompilerParams(use_tc_tiling_on_sc=False),
        scratch_types=[
            pltpu.VMEM((NB, EB), jnp.int32),
            pltpu.VMEM((NB, EB), jnp.int32),
            [pltpu.VMEM((EB, CW), jnp.bfloat16) for _ in range(NBUF)],
            pltpu.VMEM_SHARED((NPAD, CW), jnp.bfloat16),
            [pltpu.SemaphoreType.DMA for _ in range(NBUF)],
            [pltpu.SemaphoreType.DMA for _ in range(NBUF)],
        ],
    )(xs, idx16)


def _agg2_body(hs_hbm, idx_hbm, out_hbm, srcv, dstv, gbuf, acc, sem):
    cid = lax.axis_index("c")
    sid = lax.axis_index("s")
    r0 = sid * ROWS_PER_TILE
    pltpu.sync_copy(idx_hbm.at[0].at[sid].at[pl.ds(0, NB)], srcv)
    pltpu.sync_copy(idx_hbm.at[1].at[sid].at[pl.ds(0, NB)], dstv)
    pltpu.sync_copy(hs_hbm.at[cid].at[pl.ds(r0, ROWS_PER_TILE)],
                    acc.at[pl.ds(r0, ROWS_PER_TILE)])
    plsc.subcore_barrier()

    def body(j, _):
        b = 2 * j + cid
        pltpu.async_copy(hs_hbm.at[0].at[srcv.at[b]], gbuf, sem).wait()
        pltpu.sync_copy(gbuf, acc.at[dstv.at[b]], add=True)
        return 0
    lax.fori_loop(0, NBH, body, 0)
    plsc.subcore_barrier()
    pltpu.sync_copy(acc.at[pl.ds(r0, ROWS_PER_TILE)],
                    out_hbm.at[cid].at[pl.ds(r0, ROWS_PER_TILE)])


def _agg2_call(hs2, idx16):
    return pl.kernel(
        _agg2_body,
        out_type=jax.ShapeDtypeStruct((NC, NPAD, OPAD), jnp.float32),
        mesh=_mesh(),
        compiler_params=pltpu.CompilerParams(use_tc_tiling_on_sc=False),
        scratch_types=[
            pltpu.VMEM((NB, EB), jnp.int32),
            pltpu.VMEM((NB, EB), jnp.int32),
            pltpu.VMEM((EB, OPAD), jnp.float32),
            pltpu.VMEM_SHARED((NPAD, OPAD), jnp.float32),
            pltpu.SemaphoreType.DMA,
        ],
    )(hs2, idx16)


def _dinv(degb):
    deg = degb[0][:, 0:1] + degb[1][:, 0:1] + 1.0
    return lax.rsqrt(deg)


def _mm1_body(xb, wb, degb, ob):
    dinv = _dinv(degb)
    xbf = xb[...].astype(jnp.bfloat16)
    wbf = wb[...].astype(jnp.bfloat16)
    y = jnp.dot(xbf, wbf, preferred_element_type=jnp.float32)
    ys = (jnp.pad(y, ((0, 0), (0, HPAD - D_HID))) * dinv).astype(jnp.bfloat16)
    for ck in range(NCHUNK):
        ob[ck] = ys[:, ck * CW:(ck + 1) * CW]


def _mm1_call(xp, w1p, degp):
    return pl.pallas_call(
        _mm1_body,
        grid=(NPAD // RB,),
        in_specs=[
            pl.BlockSpec((RB, D_IN), lambda i: (i, 0)),
            pl.BlockSpec((D_IN, D_HID), lambda i: (0, 0)),
            pl.BlockSpec((NC, RB, OPAD), lambda i: (0, i, 0)),
        ],
        out_specs=pl.BlockSpec((NCHUNK, RB, CW), lambda i: (0, i, 0)),
        out_shape=jax.ShapeDtypeStruct((NCHUNK, NPAD, CW), jnp.bfloat16),
    )(xp, w1p, degp)


def _layer2_body(aggb, degb, b1b, w2b, ob):
    dinv = _dinv(degb)
    acc = jnp.zeros((RB, OPAD), jnp.float32)
    for ck in range(NCHUNK):
        h = jnp.maximum(aggb[ck].astype(jnp.float32) * dinv + b1b[ck], 0.0)
        acc = acc + jnp.dot(h, w2b[ck],
                            preferred_element_type=jnp.float32)
    ob[0] = acc * dinv
    ob[1] = jnp.zeros((RB, OPAD), jnp.float32)


def _layer2_call(agg1, degp, b1p, w2p):
    return pl.pallas_call(
        _layer2_body,
        grid=(NPAD // RB,),
        in_specs=[
            pl.BlockSpec((NCHUNK, RB, CW), lambda i: (0, i, 0)),
            pl.BlockSpec((NC, RB, OPAD), lambda i: (0, i, 0)),
            pl.BlockSpec((NCHUNK, 1, CW), lambda i: (0, 0, 0)),
            pl.BlockSpec((NCHUNK, CW, OPAD), lambda i: (0, 0, 0)),
        ],
        out_specs=pl.BlockSpec((NC, RB, OPAD), lambda i: (0, i, 0)),
        out_shape=jax.ShapeDtypeStruct((NC, NPAD, OPAD), jnp.float32),
    )(agg1, degp, b1p, w2p)


def _final_body(aggb, degb, b2b, ob):
    dinv = _dinv(degb)
    z = (aggb[0] + aggb[1]) * dinv + b2b[0:1, :]
    col = lax.broadcasted_iota(jnp.int32, (RB, OPAD), 1)
    z = jnp.where(col < D_OUT, z, -jnp.inf)
    m = jnp.max(z, axis=1, keepdims=True)
    s = jnp.sum(jnp.exp(z - m), axis=1, keepdims=True)
    ob[...] = (z - (m + jnp.log(s)))[:, :D_OUT]


def _final_call(agg2, degp, b2p):
    return pl.pallas_call(
        _final_body,
        grid=(NPAD // RB,),
        in_specs=[
            pl.BlockSpec((NC, RB, OPAD), lambda i: (0, i, 0)),
            pl.BlockSpec((NC, RB, OPAD), lambda i: (0, i, 0)),
            pl.BlockSpec((1, OPAD), lambda i: (0, 0)),
        ],
        out_specs=pl.BlockSpec((RB, D_OUT), lambda i: (i, 0)),
        out_shape=jax.ShapeDtypeStruct((NPAD, D_OUT), jnp.float32),
    )(agg2, degp, b2p)


def kernel(x, edge, W1, b1, W2, b2):
    edge = edge.astype(jnp.int32)
    idx16 = _fmt_call(edge)

    b1p = jnp.pad(b1, (0, HPAD - D_HID)).reshape(NCHUNK, 1, CW)
    w2p = jnp.pad(W2, ((0, HPAD - D_HID), (0, OPAD - D_OUT)))
    w2p = w2p.reshape(NCHUNK, CW, OPAD)
    b2p = jnp.pad(b2, (0, OPAD - D_OUT)).reshape(1, OPAD)
    zeros_no = jnp.zeros((NPAD, OPAD), jnp.float32)

    degp = _deg_call(idx16, zeros_no)
    xs = _mm1_call(x, W1, degp)
    agg1 = _agg1_call(xs, idx16)
    hs2 = _layer2_call(agg1, degp, b1p, w2p)
    agg2 = _agg2_call(hs2, idx16)
    out = _final_call(agg2, degp, b2p)
    return out[:N]
